# baseline (speedup 1.0000x reference)
import functools

import jax
import jax.numpy as jnp
from jax import lax
from jax.experimental import pallas as pl
from jax.experimental.pallas import tpu as pltpu
from jax.experimental.pallas import tpu_sc as plsc

N = 10000
E = 320000
D_IN = 128
H = 64

NC = 2
NS = 16
TILES = NC * NS
CH = 128
NPAD = 10240
RPS = NPAD // NS
DROWS = NPAD // 16
DRPS = DROWS // NS


def _make_seg(with_deg: bool, CPT: int, POOL: int):
    FLT = POOL // 2
    assert CPT % POOL == 0
    mesh = plsc.VectorSubcoreMesh(core_axis_name="c", subcore_axis_name="s")
    out_type = [jax.ShapeDtypeStruct((NC, NPAD, H), jnp.float32)]
    if with_deg:
        out_type.append(jax.ShapeDtypeStruct((NC, NPAD, 16), jnp.float32))
    scratch = [
        pltpu.VMEM_SHARED((NPAD, H), jnp.float32),
        pltpu.VMEM((CPT, CH), jnp.int32),
        pltpu.VMEM((CPT, CH), jnp.int32),
        pltpu.VMEM((CH, H), jnp.float32),
        pltpu.SemaphoreType.DMA,
    ]
    if with_deg:
        scratch += [
            pltpu.VMEM_SHARED((NPAD, 16), jnp.float32),
            pltpu.VMEM((CH, 16), jnp.float32),
        ]

    def body(*refs):
        if with_deg:
            (y_hbm, srcr, dstr, acc_out, deg_out,
             acc_s, src_v, dst_v, rows_v, gsem, deg_s, ones_v) = refs
        else:
            (y_hbm, srcr, dstr, acc_out,
             acc_s, src_v, dst_v, rows_v, gsem) = refs
        c = lax.axis_index("c")
        s = lax.axis_index("s")
        wid = c * NS + s

        def _zrow(i, carry):
            for k in range(H // 16):
                rows_v[i, pl.ds(k * 16, 16)] = jnp.zeros((16,), jnp.float32)
            return carry
        lax.fori_loop(0, CH, _zrow, 0)
        for b in range(RPS // CH):
            pltpu.sync_copy(rows_v,
                            acc_s.at[pl.ds(s * RPS + b * CH, CH)])

        pltpu.sync_copy(srcr.at[wid], src_v)
        pltpu.sync_copy(dstr.at[wid], dst_v)

        if with_deg:
            def _zd(i, carry):
                ones_v[i] = jnp.zeros((16,), jnp.float32)
                return carry
            lax.fori_loop(0, CH, _zd, 0)
            for b in range(RPS // CH):
                pltpu.sync_copy(ones_v, deg_s.at[pl.ds(s * RPS + b * CH, CH)])

            def _od(i, carry):
                ones_v[i] = jnp.ones((16,), jnp.float32)
                return carry
            lax.fori_loop(0, CH, _od, 0)

        plsc.subcore_barrier()

        def step(j, carry):
            pltpu.async_copy(y_hbm.at[src_v.at[j]], rows_v,
                             gsem).wait()
            pltpu.sync_copy(rows_v, acc_s.at[dst_v.at[j]], add=True)
            if with_deg:
                pltpu.sync_copy(ones_v, deg_s.at[dst_v.at[j]], add=True)
            return carry
        lax.fori_loop(0, CPT, step, 0)

        plsc.subcore_barrier()

        pltpu.sync_copy(acc_s.at[pl.ds(s * RPS, RPS)],
                        acc_out.at[c, pl.ds(s * RPS, RPS)])
        if with_deg:
            pltpu.sync_copy(deg_s.at[pl.ds(s * RPS, RPS)],
                            deg_out.at[c, pl.ds(s * RPS, RPS)])

    ot = tuple(out_type) if with_deg else out_type[0]
    return pl.kernel(
        body, out_type=ot, mesh=mesh, scratch_types=scratch,
        compiler_params=pltpu.CompilerParams(use_tc_tiling_on_sc=False))


CPT1 = 79
CPT2 = 79
_seg_deg = _make_seg(True, CPT1, 1)
_seg = _make_seg(False, CPT2, 1)


def _dense_a_body(x, w1lT, w1rT, y1, xr):
    pad = jnp.zeros((NPAD - N, H), jnp.float32)
    xv = x[...]
    y1[...] = jnp.concatenate(
        [jnp.dot(xv, w1lT[...], preferred_element_type=jnp.float32), pad], 0)
    xr[...] = jnp.concatenate(
        [jnp.dot(xv, w1rT[...], preferred_element_type=jnp.float32), pad], 0)


_dense_a = pl.pallas_call(
    _dense_a_body,
    out_shape=(jax.ShapeDtypeStruct((NPAD, H), jnp.float32),
               jax.ShapeDtypeStruct((NPAD, H), jnp.float32)),
)


def _dense_b_body(acc0, acc1, dref, xr, b1, w2lT, w2rT,
                  h1_o, y2_o, hr2_o, rdeg_o):
    dsum = (jnp.sum(dref[0], axis=1, keepdims=True)
            + jnp.sum(dref[1], axis=1, keepdims=True))
    rdeg = 16.0 / jnp.maximum(dsum, 16.0)
    h1 = jnp.maximum((acc0[...] + acc1[...]) * rdeg + xr[...] + b1[...], 0.0)
    h1_o[...] = h1
    y2_o[...] = jnp.dot(h1, w2lT[...], preferred_element_type=jnp.float32)
    hr2_o[...] = jnp.dot(h1, w2rT[...], preferred_element_type=jnp.float32)
    rdeg_o[...] = rdeg


_dense_b = pl.pallas_call(
    _dense_b_body,
    out_shape=(jax.ShapeDtypeStruct((NPAD, H), jnp.float32),
               jax.ShapeDtypeStruct((NPAD, H), jnp.float32),
               jax.ShapeDtypeStruct((NPAD, H), jnp.float32),
               jax.ShapeDtypeStruct((NPAD, 1), jnp.float32)),
)


def _dense_c_body(acc0, acc1, rdeg, hr2, b2, h1, wa, wb, bl, out_o):
    h2 = jnp.maximum((acc0[...] + acc1[...]) * rdeg[...] + hr2[...] + b2[...],
                     0.0)
    z = (jnp.sum(h1[...] * wa[...], axis=1, keepdims=True)
         + jnp.sum(h2 * wb[...], axis=1, keepdims=True) + bl[...])
    out_o[...] = jax.nn.sigmoid(z)


_dense_c = pl.pallas_call(
    _dense_c_body,
    out_shape=jax.ShapeDtypeStruct((NPAD, 1), jnp.float32),
)


def kernel(x, edge_index, W1l, b1l, W1r, W2l, b2l, W2r, Wlin, blin):
    src = edge_index[0]
    dst = edge_index[1]
    ep = TILES * CPT1 * CH
    srcp = jnp.pad(src, (0, ep - E)).reshape(TILES, CPT1, CH)
    dstp = jnp.pad(dst, (0, ep - E),
                   constant_values=NPAD - 1).reshape(TILES, CPT1, CH)

    y1, xr = _dense_a(x, W1l.T, W1r.T)
    acc1, deg = _seg_deg(y1, srcp, dstp)
    h1, y2, hr2, rdeg = _dense_b(acc1[0], acc1[1], deg, xr,
                                 b1l.reshape(1, H), W2l.T, W2r.T)
    acc2 = _seg(y2, srcp, dstp)
    out = _dense_c(acc2[0], acc2[1], rdeg, hr2, b2l.reshape(1, H), h1,
                   Wlin[:, :H], Wlin[:, H:], blin.reshape(1, 1))
    return out.reshape(NPAD)[:N]


# speedup vs baseline: 1.0098x; 1.0098x over previous
import functools

import jax
import jax.numpy as jnp
from jax import lax
from jax.experimental import pallas as pl
from jax.experimental.pallas import tpu as pltpu
from jax.experimental.pallas import tpu_sc as plsc

N = 10000
E = 320000
D_IN = 128
H = 64

NC = 2
NS = 16
TILES = NC * NS
CH = 128
NPAD = 10240
RPS = NPAD // NS
DROWS = NPAD // 16
DRPS = DROWS // NS


def _make_seg(with_deg: bool, CPT: int, POOL: int):
    FLT = POOL // 2
    assert CPT % POOL == 0
    mesh = plsc.VectorSubcoreMesh(core_axis_name="c", subcore_axis_name="s")
    out_type = [jax.ShapeDtypeStruct((NC, NPAD, H), jnp.float32)]
    if with_deg:
        out_type.append(jax.ShapeDtypeStruct((NC, NPAD, 16), jnp.float32))
    scratch = [
        pltpu.VMEM_SHARED((NPAD, H), jnp.float32),
        pltpu.VMEM((CPT, CH), jnp.int32),
        pltpu.VMEM((CPT, CH), jnp.int32),
        pltpu.VMEM((CH, H), jnp.float32),
        pltpu.SemaphoreType.DMA,
    ]
    if with_deg:
        scratch += [
            pltpu.VMEM_SHARED((NPAD, 16), jnp.float32),
            pltpu.VMEM((CH, 16), jnp.float32),
        ]

    def body(*refs):
        if with_deg:
            (y_hbm, srcr, dstr, acc_out, deg_out,
             acc_s, src_v, dst_v, rows_v, gsem, deg_s, ones_v) = refs
        else:
            (y_hbm, srcr, dstr, acc_out,
             acc_s, src_v, dst_v, rows_v, gsem) = refs
        c = lax.axis_index("c")
        s = lax.axis_index("s")
        wid = c * NS + s

        def _zrow(i, carry):
            for k in range(H // 16):
                rows_v[i, pl.ds(k * 16, 16)] = jnp.zeros((16,), jnp.float32)
            return carry
        lax.fori_loop(0, CH, _zrow, 0)
        for b in range(RPS // CH):
            pltpu.sync_copy(rows_v,
                            acc_s.at[pl.ds(s * RPS + b * CH, CH)])

        pltpu.sync_copy(srcr.at[wid], src_v)
        pltpu.sync_copy(dstr.at[wid], dst_v)

        if with_deg:
            def _zd(i, carry):
                ones_v[i] = jnp.zeros((16,), jnp.float32)
                return carry
            lax.fori_loop(0, CH, _zd, 0)
            for b in range(RPS // CH):
                pltpu.sync_copy(ones_v, deg_s.at[pl.ds(s * RPS + b * CH, CH)])

            def _od(i, carry):
                ones_v[i] = jnp.ones((16,), jnp.float32)
                return carry
            lax.fori_loop(0, CH, _od, 0)

        plsc.subcore_barrier()

        def step(j, carry):
            pltpu.async_copy(y_hbm.at[src_v.at[j]], rows_v,
                             gsem).wait()
            pltpu.sync_copy(rows_v, acc_s.at[dst_v.at[j]], add=True)
            if with_deg:
                pltpu.sync_copy(ones_v, deg_s.at[dst_v.at[j]], add=True)
            return carry
        lax.fori_loop(0, CPT, step, 0)

        plsc.subcore_barrier()

        pltpu.sync_copy(acc_s.at[pl.ds(s * RPS, RPS)],
                        acc_out.at[c, pl.ds(s * RPS, RPS)])
        if with_deg:
            pltpu.sync_copy(deg_s.at[pl.ds(s * RPS, RPS)],
                            deg_out.at[c, pl.ds(s * RPS, RPS)])

    ot = tuple(out_type) if with_deg else out_type[0]
    return pl.kernel(
        body, out_type=ot, mesh=mesh, scratch_types=scratch,
        compiler_params=pltpu.CompilerParams(use_tc_tiling_on_sc=False))


CPT1 = 79
CPT2 = 79
_seg_deg = _make_seg(True, CPT1, 1)
_seg = _make_seg(False, CPT2, 1)


def _dense_a_body(xp, w1lT, w1rT, y1, xr):
    xv = xp[...]
    y1[...] = jnp.dot(xv, w1lT[...], preferred_element_type=jnp.float32)
    xr[...] = jnp.dot(xv, w1rT[...], preferred_element_type=jnp.float32)


_dense_a = pl.pallas_call(
    _dense_a_body,
    out_shape=(jax.ShapeDtypeStruct((NPAD, H), jnp.float32),
               jax.ShapeDtypeStruct((NPAD, H), jnp.float32)),
)


def _dense_b_body(acc0, acc1, dref, xr, b1, w2lT, w2rT,
                  h1_o, y2_o, hr2_o, rdeg_o):
    dsum = (jnp.sum(dref[0], axis=1, keepdims=True)
            + jnp.sum(dref[1], axis=1, keepdims=True))
    rdeg = 16.0 / jnp.maximum(dsum, 16.0)
    h1 = jnp.maximum((acc0[...] + acc1[...]) * rdeg + xr[...] + b1[...], 0.0)
    h1_o[...] = h1
    y2_o[...] = jnp.dot(h1, w2lT[...], preferred_element_type=jnp.float32)
    hr2_o[...] = jnp.dot(h1, w2rT[...], preferred_element_type=jnp.float32)
    rdeg_o[...] = rdeg


_dense_b = pl.pallas_call(
    _dense_b_body,
    out_shape=(jax.ShapeDtypeStruct((NPAD, H), jnp.float32),
               jax.ShapeDtypeStruct((NPAD, H), jnp.float32),
               jax.ShapeDtypeStruct((NPAD, H), jnp.float32),
               jax.ShapeDtypeStruct((NPAD, 1), jnp.float32)),
)


def _dense_c_body(acc0, acc1, rdeg, hr2, b2, h1, wa, wb, bl, out_o):
    h2 = jnp.maximum((acc0[...] + acc1[...]) * rdeg[...] + hr2[...] + b2[...],
                     0.0)
    z = (jnp.sum(h1[...] * wa[...], axis=1, keepdims=True)
         + jnp.sum(h2 * wb[...], axis=1, keepdims=True) + bl[...])
    out_o[...] = jax.nn.sigmoid(z)


_dense_c = pl.pallas_call(
    _dense_c_body,
    out_shape=jax.ShapeDtypeStruct((NPAD, 1), jnp.float32),
)


def kernel(x, edge_index, W1l, b1l, W1r, W2l, b2l, W2r, Wlin, blin):
    src = edge_index[0]
    dst = edge_index[1]
    ep = TILES * CPT1 * CH
    srcp = jnp.pad(src, (0, ep - E)).reshape(TILES, CPT1, CH)
    dstp = jnp.pad(dst, (0, ep - E),
                   constant_values=NPAD - 1).reshape(TILES, CPT1, CH)

    xp = jnp.pad(x, ((0, NPAD - N), (0, 0)))
    y1, xr = _dense_a(xp, W1l.T, W1r.T)
    acc1, deg = _seg_deg(y1, srcp, dstp)
    h1, y2, hr2, rdeg = _dense_b(acc1[0], acc1[1], deg, xr,
                                 b1l.reshape(1, H), W2l.T, W2r.T)
    acc2 = _seg(y2, srcp, dstp)
    out = _dense_c(acc2[0], acc2[1], rdeg, hr2, b2l.reshape(1, H), h1,
                   Wlin[:, :H], Wlin[:, H:], blin.reshape(1, 1))
    return out.reshape(NPAD)[:N]


# speedup vs baseline: 1.3573x; 1.3442x over previous
import functools

import jax
import jax.numpy as jnp
from jax import lax
from jax.experimental import pallas as pl
from jax.experimental.pallas import tpu as pltpu
from jax.experimental.pallas import tpu_sc as plsc

N = 10000
E = 320000
D_IN = 128
H = 64

NC = 2
NS = 16
TILES = NC * NS
CH = 128
NPAD = 10240
RPS = NPAD // NS
DROWS = NPAD // 16
DRPS = DROWS // NS


def _make_seg(with_deg: bool, CPT: int, acc_dtype):
    LN = 32 if acc_dtype == jnp.bfloat16 else 16
    mesh = plsc.VectorSubcoreMesh(core_axis_name="c", subcore_axis_name="s")
    out_type = [jax.ShapeDtypeStruct((NC, NPAD, H), acc_dtype)]
    if with_deg:
        out_type.append(jax.ShapeDtypeStruct((NC, NPAD, 16), jnp.float32))
    scratch = [
        pltpu.VMEM_SHARED((NPAD, H), acc_dtype),
        pltpu.VMEM((CPT, CH), jnp.int32),
        pltpu.VMEM((CPT, CH), jnp.int32),
        pltpu.VMEM((CH, H), acc_dtype),
        pltpu.SemaphoreType.DMA,
    ]
    if with_deg:
        scratch += [
            pltpu.VMEM_SHARED((NPAD, 16), jnp.float32),
            pltpu.VMEM((CH, 16), jnp.float32),
        ]

    def body(*refs):
        if with_deg:
            (y_hbm, srcr, dstr, acc_out, deg_out,
             acc_s, src_v, dst_v, rows_v, gsem, deg_s, ones_v) = refs
        else:
            (y_hbm, srcr, dstr, acc_out,
             acc_s, src_v, dst_v, rows_v, gsem) = refs
        c = lax.axis_index("c")
        s = lax.axis_index("s")
        wid = c * NS + s

        def _zrow(i, carry):
            for k in range(H // LN):
                rows_v[i, pl.ds(k * LN, LN)] = jnp.zeros((LN,), acc_dtype)
            return carry
        lax.fori_loop(0, CH, _zrow, 0)
        for b in range(RPS // CH):
            pltpu.sync_copy(rows_v,
                            acc_s.at[pl.ds(s * RPS + b * CH, CH)])

        pltpu.sync_copy(srcr.at[wid], src_v)
        pltpu.sync_copy(dstr.at[wid], dst_v)

        if with_deg:
            def _zd(i, carry):
                ones_v[i] = jnp.zeros((16,), jnp.float32)
                return carry
            lax.fori_loop(0, CH, _zd, 0)
            for b in range(RPS // CH):
                pltpu.sync_copy(ones_v, deg_s.at[pl.ds(s * RPS + b * CH, CH)])

            def _od(i, carry):
                ones_v[i] = jnp.ones((16,), jnp.float32)
                return carry
            lax.fori_loop(0, CH, _od, 0)

        plsc.subcore_barrier()

        def step(j, carry):
            pltpu.async_copy(y_hbm.at[src_v.at[j]], rows_v,
                             gsem).wait()
            pltpu.sync_copy(rows_v, acc_s.at[dst_v.at[j]], add=True)
            if with_deg:
                pltpu.sync_copy(ones_v, deg_s.at[dst_v.at[j]], add=True)
            return carry
        lax.fori_loop(0, CPT, step, 0)

        plsc.subcore_barrier()

        pltpu.sync_copy(acc_s.at[pl.ds(s * RPS, RPS)],
                        acc_out.at[c, pl.ds(s * RPS, RPS)])
        if with_deg:
            pltpu.sync_copy(deg_s.at[pl.ds(s * RPS, RPS)],
                            deg_out.at[c, pl.ds(s * RPS, RPS)])

    ot = tuple(out_type) if with_deg else out_type[0]
    return pl.kernel(
        body, out_type=ot, mesh=mesh, scratch_types=scratch,
        compiler_params=pltpu.CompilerParams(use_tc_tiling_on_sc=False))


CPT1 = 79
_seg_deg = _make_seg(True, CPT1, jnp.bfloat16)
_seg = _make_seg(False, CPT1, jnp.bfloat16)


def _dense_a_body(xp, w1lT, w1rT, y1, xr):
    xv = xp[...]
    y1[...] = jnp.dot(xv, w1lT[...],
                      preferred_element_type=jnp.float32).astype(jnp.bfloat16)
    xr[...] = jnp.dot(xv, w1rT[...], preferred_element_type=jnp.float32)


_dense_a = pl.pallas_call(
    _dense_a_body,
    out_shape=(jax.ShapeDtypeStruct((NPAD, H), jnp.bfloat16),
               jax.ShapeDtypeStruct((NPAD, H), jnp.float32)),
)


def _dense_b_body(acc0, acc1, d0, d1, xr, b1, w2lT, w2rT,
                  h1_o, y2_o, hr2_o, rdeg_o):
    rdeg = 1.0 / jnp.maximum(d0[...] + d1[...], 1.0)
    agg = acc0[...].astype(jnp.float32) + acc1[...].astype(jnp.float32)
    h1 = jnp.maximum(agg * rdeg + xr[...] + b1[...], 0.0)
    h1_o[...] = h1
    y2_o[...] = jnp.dot(h1, w2lT[...],
                        preferred_element_type=jnp.float32).astype(jnp.bfloat16)
    hr2_o[...] = jnp.dot(h1, w2rT[...], preferred_element_type=jnp.float32)
    rdeg_o[...] = rdeg


_dense_b = pl.pallas_call(
    _dense_b_body,
    out_shape=(jax.ShapeDtypeStruct((NPAD, H), jnp.float32),
               jax.ShapeDtypeStruct((NPAD, H), jnp.bfloat16),
               jax.ShapeDtypeStruct((NPAD, H), jnp.float32),
               jax.ShapeDtypeStruct((NPAD, 1), jnp.float32)),
)


def _dense_c_body(acc0, acc1, rdeg, hr2, b2, h1, wa, wb, bl, out_o):
    agg = acc0[...].astype(jnp.float32) + acc1[...].astype(jnp.float32)
    h2 = jnp.maximum(agg * rdeg[...] + hr2[...] + b2[...], 0.0)
    z = (jnp.sum(h1[...] * wa[...], axis=1, keepdims=True)
         + jnp.sum(h2 * wb[...], axis=1, keepdims=True) + bl[...])
    out_o[...] = jax.nn.sigmoid(z)


_dense_c = pl.pallas_call(
    _dense_c_body,
    out_shape=jax.ShapeDtypeStruct((NPAD, 1), jnp.float32),
)


def kernel(x, edge_index, W1l, b1l, W1r, W2l, b2l, W2r, Wlin, blin):
    src = edge_index[0]
    dst = edge_index[1]
    ep = TILES * CPT1 * CH
    srcp = jnp.pad(src, (0, ep - E)).reshape(TILES, CPT1, CH)
    dstp = jnp.pad(dst, (0, ep - E),
                   constant_values=NPAD - 1).reshape(TILES, CPT1, CH)

    xp = jnp.pad(x, ((0, NPAD - N), (0, 0)))
    y1, xr = _dense_a(xp, W1l.T, W1r.T)
    acc1, deg = _seg_deg(y1, srcp, dstp)
    h1, y2, hr2, rdeg = _dense_b(acc1[0], acc1[1],
                                 deg[0, :, :1], deg[1, :, :1], xr,
                                 b1l.reshape(1, H), W2l.T, W2r.T)
    acc2 = _seg(y2, srcp, dstp)
    out = _dense_c(acc2[0], acc2[1], rdeg, hr2, b2l.reshape(1, H), h1,
                   Wlin[:, :H], Wlin[:, H:], blin.reshape(1, 1))
    return out.reshape(NPAD)[:N]
